# baseline (speedup 1.0000x reference)
import functools

import jax
import jax.numpy as jnp
from jax import lax
from jax.experimental import pallas as pl
from jax.experimental.pallas import tpu as pltpu
from jax.experimental.pallas import tpu_sc as plsc

B = 16384
D = 64
NW = 32
BPW = B // NW
SUB = 128
NSUB = BPW // SUB
NCHUNK = BPW // 16
BW = 16

_mesh = plsc.VectorSubcoreMesh(core_axis_name="c", subcore_axis_name="s")


@functools.partial(
    pl.kernel,
    mesh=_mesh,
    out_type=jax.ShapeDtypeStruct((B,), jnp.float32),
    scratch_types=[
        pltpu.VMEM((NSUB, SUB), jnp.int32),
        pltpu.VMEM((NSUB, SUB), jnp.int32),
        pltpu.VMEM((NSUB, SUB), jnp.int32),
        pltpu.VMEM((NSUB, SUB), jnp.int32),
        pltpu.VMEM((BPW, D), jnp.float32),
        pltpu.VMEM((BPW, D), jnp.float32),
        pltpu.VMEM((BPW, D), jnp.float32),
        pltpu.VMEM((BPW, BW), jnp.float32),
        pltpu.VMEM((1, D), jnp.float32),
        pltpu.VMEM((BPW,), jnp.float32),
        pltpu.SemaphoreType.DMA,
    ],
    compiler_params=pltpu.CompilerParams(
        needs_layout_passes=False, use_tc_tiling_on_sc=False),
)
def _trans_rec_sc(uids_hbm, lids_hbm, pids_hbm, ut_hbm, it_hbm, g_hbm, bias_hbm,
                  out_hbm, uid_v, lid_v, pid_v, bid_v, u_v, l_v, p_v, b_v, g_v,
                  o_v, sem):
    wid = lax.axis_index("s") * 2 + lax.axis_index("c")
    base = wid * BPW

    pltpu.sync_copy(uids_hbm.at[pl.ds(wid * NSUB, NSUB), :], uid_v)
    pltpu.sync_copy(lids_hbm.at[pl.ds(wid * NSUB, NSUB), :], lid_v)
    pltpu.sync_copy(pids_hbm.at[pl.ds(wid * NSUB, NSUB), :], pid_v)
    pltpu.sync_copy(g_hbm, g_v)

    copies = []
    for j in range(NSUB):
        dst = pl.ds(j * SUB, SUB)
        copies.append(pltpu.async_copy(ut_hbm.at[uid_v.at[j]], u_v.at[dst], sem))
        copies.append(pltpu.async_copy(it_hbm.at[lid_v.at[j]], l_v.at[dst], sem))
        copies.append(pltpu.async_copy(it_hbm.at[pid_v.at[j]], p_v.at[dst], sem))

    iota16 = lax.iota(jnp.int32, 16)
    for t in range(BPW // 16):
        flat = iota16 + (t * 16)
        r = flat >> 7
        col = flat & 127
        pid_c = plsc.load_gather(pid_v, [r, col])
        plsc.store_scatter(bid_v, [r, col], pid_c >> 4)
    for j in range(NSUB):
        dst = pl.ds(j * SUB, SUB)
        copies.append(pltpu.async_copy(bias_hbm.at[bid_v.at[j]], b_v.at[dst], sem))
    for cp in copies:
        cp.wait()

    zeros16 = jnp.zeros((16,), jnp.int32)
    for c in range(NCHUNK):
        rows = iota16 + (c * 16)

        def dim_body(d, acc):
            cols = jnp.full((16,), d, jnp.int32)
            u = plsc.load_gather(u_v, [rows, cols])
            l = plsc.load_gather(l_v, [rows, cols])
            p = plsc.load_gather(p_v, [rows, cols])
            gd = plsc.load_gather(g_v, [zeros16, cols])
            diff = (u + l) - (p - gd)
            return acc + diff * diff

        acc = lax.fori_loop(0, D, dim_body, jnp.zeros((16,), jnp.float32))

        bits = plsc.bitcast(acc, jnp.int32)
        y = plsc.bitcast(jnp.int32(0x1FBD1DF5) + (bits >> 1), jnp.float32)
        for _ in range(3):
            y = 0.5 * (y + acc / y)

        r = rows >> 7
        col = rows & 127
        pid_c = plsc.load_gather(pid_v, [r, col])
        bias = plsc.load_gather(b_v, [rows, pid_c & (BW - 1)])
        o_v[pl.ds(c * 16, 16)] = bias - y

    pltpu.sync_copy(o_v, out_hbm.at[pl.ds(base, BPW)])


def kernel(user_ids, last_items, pre_items, user_table, item_table,
           global_transition, item_biases):
    uid = user_ids.astype(jnp.int32).reshape(B // SUB, SUB)
    lid = last_items.astype(jnp.int32).reshape(B // SUB, SUB)
    pid = pre_items.astype(jnp.int32).reshape(B // SUB, SUB)
    bias16 = item_biases.reshape(-1, BW)
    return _trans_rec_sc(uid, lid, pid, user_table, item_table,
                         global_transition, bias16)
